# baseline (speedup 1.0000x reference)
import functools

import jax
import jax.numpy as jnp
from jax import lax
from jax.experimental import pallas as pl
from jax.experimental.pallas import tpu as pltpu
from jax.experimental.pallas import tpu_sc as plsc

_NC = 2
_NS = 16
_NW = _NC * _NS
_CHUNK = 128


def _sc_gather(x_grouped, token_table, n_rows, emb):
    per_w = n_rows // _NW
    nchunk = per_w // _CHUNK
    mesh = plsc.VectorSubcoreMesh(
        core_axis_name="c", subcore_axis_name="s",
        num_cores=_NC, num_subcores=_NS,
    )

    @functools.partial(
        pl.kernel,
        out_type=jax.ShapeDtypeStruct((n_rows, emb), jnp.float32),
        mesh=mesh,
        compiler_params=pltpu.CompilerParams(use_tc_tiling_on_sc=False),
        scratch_types=[
            pltpu.VMEM((nchunk, _CHUNK), jnp.int32),
            pltpu.VMEM((2, _CHUNK, emb), jnp.float32),
            pltpu.SemaphoreType.DMA,
            pltpu.SemaphoreType.DMA,
        ],
    )
    def gather_kernel(x_hbm, table_hbm, out_hbm, idx_v, rows_v, sem0, sem1):
        wid = lax.axis_index("s") * _NC + lax.axis_index("c")
        base = wid * per_w
        pltpu.sync_copy(x_hbm.at[wid], idx_v)
        sems = (sem0, sem1)
        copies = [None, None]
        copies[0] = pltpu.async_copy(
            table_hbm.at[idx_v.at[0]], rows_v.at[0], sems[0])
        for j in range(nchunk):
            cur = j % 2
            if j + 1 < nchunk:
                nxt = (j + 1) % 2
                copies[nxt] = pltpu.async_copy(
                    table_hbm.at[idx_v.at[j + 1]], rows_v.at[nxt], sems[nxt])
            copies[cur].wait()
            pltpu.sync_copy(
                rows_v.at[cur], out_hbm.at[pl.ds(base + j * _CHUNK, _CHUNK)])

    return gather_kernel(x_grouped, token_table)


def _tc_linear(h2, pos_rep, W, b2, r_blk):
    n_rows, emb = h2.shape
    vocab = W.shape[1]
    grid = (n_rows // r_blk,)

    def body(h_ref, pos_ref, w_ref, b_ref, out_ref):
        x = h_ref[...] + pos_ref[...]
        y = jnp.dot(x, w_ref[...], preferred_element_type=jnp.float32)
        out_ref[...] = y + b_ref[...]

    return pl.pallas_call(
        body,
        grid=grid,
        in_specs=[
            pl.BlockSpec((r_blk, emb), lambda i: (i, 0)),
            pl.BlockSpec((r_blk, emb), lambda i: (0, 0)),
            pl.BlockSpec((emb, vocab), lambda i: (0, 0)),
            pl.BlockSpec((1, vocab), lambda i: (0, 0)),
        ],
        out_specs=pl.BlockSpec((r_blk, vocab), lambda i: (i, 0)),
        out_shape=jax.ShapeDtypeStruct((n_rows, vocab), jnp.float32),
    )(h2, pos_rep, W, b2)


def kernel(X, token_table, pos_table, W, b):
    batch, seq = X.shape
    vocab, emb = token_table.shape
    n_rows = batch * seq
    per_w = n_rows // _NW

    x_grouped = X.astype(jnp.int32).reshape(_NW, per_w // _CHUNK, _CHUNK)
    h2 = _sc_gather(x_grouped, token_table, n_rows, emb)

    r_blk = 2560
    pos_rep = jnp.tile(pos_table, (r_blk // seq, 1))
    b2 = b.reshape(1, vocab)
    out2 = _tc_linear(h2, pos_rep, W, b2, r_blk)
    return out2.reshape(batch, seq, vocab)


# speedup vs baseline: 1.0080x; 1.0080x over previous
import functools

import jax
import jax.numpy as jnp
from jax import lax
from jax.experimental import pallas as pl
from jax.experimental.pallas import tpu as pltpu
from jax.experimental.pallas import tpu_sc as plsc

_NC = 2
_NS = 16
_NW = _NC * _NS
_CHUNK = 128
_EPAD = 128


def _sc_gather(x_grouped, table_pad, n_rows):
    per_w = n_rows // _NW
    nchunk = per_w // _CHUNK
    mesh = plsc.VectorSubcoreMesh(
        core_axis_name="c", subcore_axis_name="s",
        num_cores=_NC, num_subcores=_NS,
    )

    @functools.partial(
        pl.kernel,
        out_type=jax.ShapeDtypeStruct((n_rows, _EPAD), jnp.float32),
        mesh=mesh,
        scratch_types=[
            pltpu.VMEM((nchunk, _CHUNK), jnp.int32),
            pltpu.VMEM((2, _CHUNK, _EPAD), jnp.float32),
            pltpu.SemaphoreType.DMA,
            pltpu.SemaphoreType.DMA,
        ],
    )
    def gather_kernel(x_hbm, table_hbm, out_hbm, idx_v, rows_v, sem0, sem1):
        wid = lax.axis_index("s") * _NC + lax.axis_index("c")
        base = wid * per_w
        pltpu.sync_copy(x_hbm.at[wid], idx_v)
        sems = (sem0, sem1)
        copies = [None, None]
        copies[0] = pltpu.async_copy(
            table_hbm.at[idx_v.at[0]], rows_v.at[0], sems[0])
        for j in range(nchunk):
            cur = j % 2
            if j + 1 < nchunk:
                nxt = (j + 1) % 2
                copies[nxt] = pltpu.async_copy(
                    table_hbm.at[idx_v.at[j + 1]], rows_v.at[nxt], sems[nxt])
            copies[cur].wait()
            pltpu.sync_copy(
                rows_v.at[cur], out_hbm.at[pl.ds(base + j * _CHUNK, _CHUNK)])

    return gather_kernel(x_grouped, table_pad)


def _tc_linear(h2, pos_rep, w_pad, b2, r_blk):
    n_rows = h2.shape[0]
    vocab = w_pad.shape[1]
    grid = (n_rows // r_blk,)

    def body(h_ref, pos_ref, w_ref, b_ref, out_ref):
        x = h_ref[...] + pos_ref[...]
        y = jnp.dot(x, w_ref[...], preferred_element_type=jnp.float32)
        out_ref[...] = y + b_ref[...]

    return pl.pallas_call(
        body,
        grid=grid,
        in_specs=[
            pl.BlockSpec((r_blk, _EPAD), lambda i: (i, 0)),
            pl.BlockSpec((r_blk, _EPAD), lambda i: (0, 0)),
            pl.BlockSpec((_EPAD, vocab), lambda i: (0, 0)),
            pl.BlockSpec((1, vocab), lambda i: (0, 0)),
        ],
        out_specs=pl.BlockSpec((r_blk, vocab), lambda i: (i, 0)),
        out_shape=jax.ShapeDtypeStruct((n_rows, vocab), jnp.float32),
        compiler_params=pltpu.CompilerParams(
            dimension_semantics=("parallel",)),
    )(h2, pos_rep, w_pad, b2)


def kernel(X, token_table, pos_table, W, b):
    batch, seq = X.shape
    vocab, emb = token_table.shape
    vocab_out = W.shape[1]
    n_rows = batch * seq
    per_w = n_rows // _NW

    x_grouped = X.astype(jnp.int32).reshape(_NW, per_w // _CHUNK, _CHUNK)
    table_pad = jnp.pad(token_table, ((0, 0), (0, _EPAD - emb)))
    h2 = _sc_gather(x_grouped, table_pad, n_rows)

    r_blk = 1280
    pos_rep = jnp.tile(jnp.pad(pos_table, ((0, 0), (0, _EPAD - emb))),
                       (r_blk // seq, 1))
    w_pad = jnp.pad(W, ((0, _EPAD - emb), (0, 0)))
    b2 = b.reshape(1, vocab_out)
    out2 = _tc_linear(h2, pos_rep, w_pad, b2, r_blk)
    return out2.reshape(batch, seq, vocab_out)


# speedup vs baseline: 1.4529x; 1.4414x over previous
import functools

import jax
import jax.numpy as jnp
from jax import lax
from jax.experimental import pallas as pl
from jax.experimental.pallas import tpu as pltpu
from jax.experimental.pallas import tpu_sc as plsc

_NC = 2
_NS = 16
_NW = _NC * _NS
_CHUNK = 128
_EPAD = 128


def _sc_gather(x_grouped, table_pad, n_rows):
    per_w = n_rows // _NW
    nchunk = per_w // _CHUNK
    mesh = plsc.VectorSubcoreMesh(
        core_axis_name="c", subcore_axis_name="s",
        num_cores=_NC, num_subcores=_NS,
    )

    @functools.partial(
        pl.kernel,
        out_type=jax.ShapeDtypeStruct((n_rows, _EPAD), jnp.float32),
        mesh=mesh,
        scratch_types=[
            pltpu.VMEM((nchunk, _CHUNK), jnp.int32),
            pltpu.VMEM((2, _CHUNK, _EPAD), jnp.float32),
            pltpu.SemaphoreType.DMA,
            pltpu.SemaphoreType.DMA,
        ],
    )
    def gather_kernel(x_hbm, table_hbm, out_hbm, idx_v, rows_v, sem0, sem1):
        wid = lax.axis_index("s") * _NC + lax.axis_index("c")
        base = wid * per_w
        pltpu.sync_copy(x_hbm.at[wid], idx_v)
        sems = (sem0, sem1)
        copies = [None, None]
        copies[0] = pltpu.async_copy(
            table_hbm.at[idx_v.at[0]], rows_v.at[0], sems[0])
        for j in range(nchunk):
            cur = j % 2
            if j + 1 < nchunk:
                nxt = (j + 1) % 2
                copies[nxt] = pltpu.async_copy(
                    table_hbm.at[idx_v.at[j + 1]], rows_v.at[nxt], sems[nxt])
            copies[cur].wait()
            pltpu.sync_copy(
                rows_v.at[cur], out_hbm.at[pl.ds(base + j * _CHUNK, _CHUNK)])

    return gather_kernel(x_grouped, table_pad)


def _tc_linear(h2, pos_rep, w_pad, b2, batch, seq, b_blk):
    vocab = w_pad.shape[1]
    r_blk = b_blk * seq
    grid = (batch // b_blk,)

    def body(h_ref, pos_ref, w_ref, b_ref, out_ref):
        x = h_ref[...] + pos_ref[...]
        y = jnp.dot(x, w_ref[...], preferred_element_type=jnp.float32)
        out_ref[...] = y.reshape(b_blk, seq, vocab) + b_ref[...]

    return pl.pallas_call(
        body,
        grid=grid,
        in_specs=[
            pl.BlockSpec((r_blk, _EPAD), lambda i: (i, 0)),
            pl.BlockSpec((r_blk, _EPAD), lambda i: (0, 0)),
            pl.BlockSpec((_EPAD, vocab), lambda i: (0, 0)),
            pl.BlockSpec((1, 1, vocab), lambda i: (0, 0, 0)),
        ],
        out_specs=pl.BlockSpec((b_blk, seq, vocab), lambda i: (i, 0, 0)),
        out_shape=jax.ShapeDtypeStruct((batch, seq, vocab), jnp.float32),
        compiler_params=pltpu.CompilerParams(
            dimension_semantics=("parallel",)),
    )(h2, pos_rep, w_pad, b2)


def kernel(X, token_table, pos_table, W, b):
    batch, seq = X.shape
    vocab, emb = token_table.shape
    vocab_out = W.shape[1]
    n_rows = batch * seq
    per_w = n_rows // _NW

    x_grouped = X.astype(jnp.int32).reshape(_NW, per_w // _CHUNK, _CHUNK)
    table_pad = jnp.pad(token_table, ((0, 0), (0, _EPAD - emb)))
    h2 = _sc_gather(x_grouped, table_pad, n_rows)

    b_blk = 64
    pos_rep = jnp.tile(jnp.pad(pos_table, ((0, 0), (0, _EPAD - emb))),
                       (b_blk, 1))
    w_pad = jnp.pad(W, ((0, _EPAD - emb), (0, 0)))
    b3 = b.reshape(1, 1, vocab_out)
    return _tc_linear(h2, pos_rep, w_pad, b3, batch, seq, b_blk)
